# Initial kernel scaffold; baseline (speedup 1.0000x reference)
import jax
import jax.numpy as jnp
from jax.experimental import pallas as pl


def kernel(adj, n_feat, W1, b1, W2, b2):
    raise NotImplementedError("write your pallas kernel here")


# speedup vs baseline: 1.5673x; 1.5673x over previous
import jax
import jax.numpy as jnp
from jax.experimental import pallas as pl


def _gcn_body(adj_ref, feat_ref, w1_ref, b1_ref, w2_ref, b2_ref, out_ref):
    adj = adj_ref[0]
    feat = feat_ref[0]
    s = adj.shape[0]
    deg = jnp.maximum(jnp.sum(adj, axis=1, keepdims=True), 1.0)
    invdeg = 1.0 / deg
    g = jnp.dot(feat, w1_ref[...], preferred_element_type=jnp.float32)
    m = jnp.dot(adj, g, preferred_element_type=jnp.float32)
    h = jnp.maximum(m * invdeg + b1_ref[...], 0.0)
    u = jnp.dot(adj, h, preferred_element_type=jnp.float32)
    y = jnp.sum(u * invdeg, axis=0, keepdims=True) * (1.0 / s)
    y = jnp.dot(y, w2_ref[...], preferred_element_type=jnp.float32) + b2_ref[...]
    nrm = jnp.sqrt(jnp.sum(y * y))
    out_ref[0] = y / jnp.maximum(nrm, 1e-5)


@jax.jit
def kernel(adj, n_feat, W1, b1, W2, b2):
    B, S, _ = adj.shape
    FT = n_feat.shape[-1]
    H = W1.shape[-1]
    O = W2.shape[-1]
    b1r = b1.reshape(1, H)
    b2r = b2.reshape(1, O)
    return pl.pallas_call(
        _gcn_body,
        grid=(B,),
        in_specs=[
            pl.BlockSpec((1, S, S), lambda b: (b, 0, 0)),
            pl.BlockSpec((1, S, FT), lambda b: (b, 0, 0)),
            pl.BlockSpec((FT, H), lambda b: (0, 0)),
            pl.BlockSpec((1, H), lambda b: (0, 0)),
            pl.BlockSpec((H, O), lambda b: (0, 0)),
            pl.BlockSpec((1, O), lambda b: (0, 0)),
        ],
        out_specs=pl.BlockSpec((1, 1, O), lambda b: (b, 0, 0)),
        out_shape=jax.ShapeDtypeStruct((B, 1, O), jnp.float32),
    )(adj, n_feat, W1, b1r, W2, b2r).reshape(B, O)


# speedup vs baseline: 2.0782x; 1.3260x over previous
import jax
import jax.numpy as jnp
from jax.experimental import pallas as pl


def _gcn_body(adj_ref, feat_ref, w1_ref, b1_ref, w2_ref, b2_ref, out_ref):
    adj = adj_ref[0]
    feat = feat_ref[0]
    s = adj.shape[0]
    deg = jnp.maximum(jnp.sum(adj, axis=1, keepdims=True), 1.0)
    invdeg = 1.0 / deg
    g = jnp.dot(feat, w1_ref[...], preferred_element_type=jnp.float32)
    m = jnp.dot(adj, g, preferred_element_type=jnp.float32)
    h = jnp.maximum(m * invdeg + b1_ref[...], 0.0)
    c = jax.lax.dot_general(invdeg, adj, (((0,), (0,)), ((), ())),
                            preferred_element_type=jnp.float32)
    y = jnp.dot(c, h, preferred_element_type=jnp.float32) * (1.0 / s)
    y = jnp.dot(y, w2_ref[...], preferred_element_type=jnp.float32) + b2_ref[...]
    nrm = jnp.sqrt(jnp.sum(y * y))
    out_ref[0] = y / jnp.maximum(nrm, 1e-5)


@jax.jit
def kernel(adj, n_feat, W1, b1, W2, b2):
    B, S, _ = adj.shape
    FT = n_feat.shape[-1]
    H = W1.shape[-1]
    O = W2.shape[-1]
    b1r = b1.reshape(1, H)
    b2r = b2.reshape(1, O)
    return pl.pallas_call(
        _gcn_body,
        grid=(B,),
        in_specs=[
            pl.BlockSpec((1, S, S), lambda b: (b, 0, 0)),
            pl.BlockSpec((1, S, FT), lambda b: (b, 0, 0)),
            pl.BlockSpec((FT, H), lambda b: (0, 0)),
            pl.BlockSpec((1, H), lambda b: (0, 0)),
            pl.BlockSpec((H, O), lambda b: (0, 0)),
            pl.BlockSpec((1, O), lambda b: (0, 0)),
        ],
        out_specs=pl.BlockSpec((1, 1, O), lambda b: (b, 0, 0)),
        out_shape=jax.ShapeDtypeStruct((B, 1, O), jnp.float32),
    )(adj, n_feat, W1, b1r, W2, b2r).reshape(B, O)
